# baseline (speedup 1.0000x reference)
import jax
import jax.numpy as jnp
from jax import lax
from jax.experimental import pallas as pl
from jax.experimental.pallas import tpu as pltpu
from jax.experimental.pallas import tpu_sc as plsc

_NC = 2
_NS = 16
_NW = _NC * _NS
_C = 128
_NBUF = 10


def kernel(segment_ids, table):
    batch, hist = segment_ids.shape
    num_rows, dim = table.shape
    total = batch * hist
    per_w = total // _NW
    n_chunks = per_w // _C
    assert total % _NW == 0 and per_w % _C == 0 and n_chunks % _NBUF == 0

    flat_idx = segment_ids.reshape(total).astype(jnp.int32)
    mesh = plsc.VectorSubcoreMesh(core_axis_name="c", subcore_axis_name="s")

    @pl.kernel(
        out_type=jax.ShapeDtypeStruct((total, dim), table.dtype),
        mesh=mesh,
        scratch_types=[
            pltpu.VMEM((per_w,), jnp.int32),
            pltpu.VMEM((_NBUF, _C, dim), jnp.float32),
            pltpu.SemaphoreType.DMA((_NBUF,)),
            pltpu.SemaphoreType.DMA((_NBUF,)),
        ],
        compiler_params=pltpu.CompilerParams(use_tc_tiling_on_sc=False),
    )
    def gather_kernel(table_hbm, idx_hbm, out_hbm, idx_v, rows_v, gsem, wsem):
        wid = lax.axis_index("s") * _NC + lax.axis_index("c")
        base = wid * per_w
        pltpu.sync_copy(idx_hbm.at[pl.ds(base, per_w)], idx_v)

        def fire_gather(g, b):
            pltpu.async_copy(
                table_hbm.at[idx_v.at[pl.ds(g * _C, _C)]],
                rows_v.at[b],
                gsem.at[b],
            )

        for b in range(_NBUF):
            fire_gather(b, b)

        @pl.loop(0, n_chunks, step=_NBUF)
        def _(g0):
            for b in range(_NBUF):
                g = g0 + b
                pltpu.make_async_copy(
                    table_hbm.at[idx_v.at[pl.ds(g * _C, _C)]],
                    rows_v.at[b],
                    gsem.at[b],
                ).wait()
                wb = pltpu.async_copy(
                    rows_v.at[b],
                    out_hbm.at[pl.ds(base + g * _C, _C)],
                    wsem.at[b],
                )

                @pl.when(g + _NBUF < n_chunks)
                def _():
                    wb.wait()
                    fire_gather(g + _NBUF, b)

        for b in range(_NBUF):
            pltpu.make_async_copy(
                rows_v.at[b],
                out_hbm.at[pl.ds(base, _C)],
                wsem.at[b],
            ).wait()

    out = gather_kernel(table, flat_idx)
    return out.reshape(batch, hist, dim)


# speedup vs baseline: 1.6234x; 1.6234x over previous
import jax
import jax.numpy as jnp
from jax import lax
from jax.experimental import pallas as pl
from jax.experimental.pallas import tpu as pltpu
from jax.experimental.pallas import tpu_sc as plsc

_NC = 2
_NS = 16
_NW = _NC * _NS
_RPC = 8
_NBUF = 2


def kernel(segment_ids, table):
    batch, hist = segment_ids.shape
    num_rows, dim = table.shape
    total = batch * hist
    per_w = total // _NW
    rows_per_w = batch // _NW
    spc = _RPC * hist
    n_sc = rows_per_w // _RPC
    runs = []
    off = 0
    while off < spc:
        ln = min(128, spc - off)
        runs.append((off, ln))
        off += ln
    assert total % _NW == 0 and rows_per_w % _RPC == 0 and n_sc % _NBUF == 0
    assert all(o % 8 == 0 and l % 8 == 0 for o, l in runs)

    flat_idx = segment_ids.reshape(total).astype(jnp.int32)
    mesh = plsc.VectorSubcoreMesh(core_axis_name="c", subcore_axis_name="s")

    @pl.kernel(
        out_type=jax.ShapeDtypeStruct((batch, hist, dim), table.dtype),
        mesh=mesh,
        scratch_types=[
            pltpu.VMEM((per_w,), jnp.int32),
            pltpu.VMEM((_NBUF, spc, dim), jnp.float32),
            pltpu.SemaphoreType.DMA((_NBUF, len(runs))),
            pltpu.SemaphoreType.DMA((_NBUF, _RPC)),
        ],
        compiler_params=pltpu.CompilerParams(use_tc_tiling_on_sc=False),
    )
    def gather_kernel(table_hbm, idx_hbm, out_hbm, idx_v, rows_v, gsem, wsem):
        wid = lax.axis_index("s") * _NC + lax.axis_index("c")
        base = wid * per_w
        row_base = wid * rows_per_w
        pltpu.sync_copy(idx_hbm.at[pl.ds(base, per_w)], idx_v)

        def fire_superchunk(c, b):
            for j, (o, ln) in enumerate(runs):
                pltpu.async_copy(
                    table_hbm.at[idx_v.at[pl.ds(c * spc + o, ln)]],
                    rows_v.at[b, pl.ds(o, ln)],
                    gsem.at[b, j],
                )

        for b in range(_NBUF):
            fire_superchunk(b, b)

        @pl.loop(0, n_sc, step=_NBUF)
        def _(c0):
            for b in range(_NBUF):
                c = c0 + b
                for j, (o, ln) in enumerate(runs):
                    pltpu.make_async_copy(
                        table_hbm.at[idx_v.at[pl.ds(c * spc + o, ln)]],
                        rows_v.at[b, pl.ds(o, ln)],
                        gsem.at[b, j],
                    ).wait()
                for i in range(_RPC):
                    pltpu.async_copy(
                        rows_v.at[b, pl.ds(i * hist, hist)],
                        out_hbm.at[row_base + c * _RPC + i],
                        wsem.at[b, i],
                    )

                @pl.when(c + _NBUF < n_sc)
                def _():
                    for i in range(_RPC):
                        pltpu.make_async_copy(
                            rows_v.at[b, pl.ds(i * hist, hist)],
                            out_hbm.at[row_base],
                            wsem.at[b, i],
                        ).wait()
                    fire_superchunk(c + _NBUF, b)

        for b in range(_NBUF):
            for i in range(_RPC):
                pltpu.make_async_copy(
                    rows_v.at[b, pl.ds(i * hist, hist)],
                    out_hbm.at[row_base],
                    wsem.at[b, i],
                ).wait()

    return gather_kernel(table, flat_idx)
